# Initial kernel scaffold; baseline (speedup 1.0000x reference)
import jax
import jax.numpy as jnp
from jax.experimental import pallas as pl


def kernel(x, edge_index, perm, aug1_nodes, aug1_edge_index, aug2_nodes, aug2_edge_index, W, Wd):
    raise NotImplementedError("write your pallas kernel here")


# speedup vs baseline: 4.3407x; 4.3407x over previous
import functools

import jax
import jax.numpy as jnp
from jax import lax
from jax.experimental import pallas as pl
from jax.experimental.pallas import tpu as pltpu
from jax.experimental.pallas import tpu_sc as plsc

N = 10000
E = 320000
D = 128
H = 128
NA = 8000
EA = 204800

NC = 2
NS = 16
L = 16

K = 80
N_PAD = 10240
NA_PAD = 8192
DEG_PAD = 10240

def _sc_segment_sums(xw, src, dst, perm, a1n, a1src, a1dst, a2n, a2src, a2dst):
    mesh = plsc.VectorSubcoreMesh(core_axis_name="c", subcore_axis_name="s")

    out_type = (
        jax.ShapeDtypeStruct((N_PAD, H), jnp.float32),
        jax.ShapeDtypeStruct((N_PAD, H), jnp.float32),
        jax.ShapeDtypeStruct((NA_PAD, H), jnp.float32),
        jax.ShapeDtypeStruct((NA_PAD, H), jnp.float32),
        jax.ShapeDtypeStruct((DEG_PAD,), jnp.float32),
        jax.ShapeDtypeStruct((DEG_PAD,), jnp.float32),
        jax.ShapeDtypeStruct((DEG_PAD,), jnp.float32),
    )

    scratch = [
        pltpu.VMEM_SHARED((N_PAD, H), jnp.float32),
        pltpu.VMEM_SHARED((DEG_PAD,), jnp.float32),
        pltpu.VMEM((K,), jnp.int32),
        pltpu.VMEM((K,), jnp.int32),
        pltpu.VMEM((K,), jnp.int32),
        pltpu.VMEM((K, H), jnp.float32),
        pltpu.VMEM((K,), jnp.float32),
        pltpu.VMEM((128, H), jnp.float32),
        pltpu.VMEM((640,), jnp.float32),
        pltpu.SemaphoreType.DMA,
    ]

    @functools.partial(pl.kernel, out_type=out_type, mesh=mesh,
                       scratch_types=scratch)
    def k(xw_h, src_h, dst_h, perm_h, a1n_h, a1src_h, a1dst_h, a2n_h,
          a2src_h, a2dst_h,
          aggp_h, aggn_h, agg1_h, agg2_h, deg_h, deg1_h, deg2_h,
          acc_sh, deg_sh, src_v, dst_v, idx_v, rows_v, ones_v,
          zrow_v, zdeg_v, sem):
        cid = lax.axis_index("c")
        sid = lax.axis_index("s")
        _Z16 = jnp.zeros((L,), jnp.float32)
        _O16 = jnp.ones((L,), jnp.float32)

        def _zr(i, c):
            for j in range(H // L):
                zrow_v[i, pl.ds(j * L, L)] = _Z16
            return c
        lax.fori_loop(0, 128, _zr, 0)
        for j in range(640 // L):
            zdeg_v[pl.ds(j * L, L)] = _Z16
        for j in range(K // L):
            ones_v[pl.ds(j * L, L)] = _O16

        def zero_acc(nrows_per_tile):
            r0 = sid * nrows_per_tile
            for j in range(nrows_per_tile // 128):
                pltpu.sync_copy(zrow_v, acc_sh.at[pl.ds(r0 + j * 128, 128)])
            pltpu.sync_copy(zdeg_v, deg_sh.at[pl.ds(sid * 640, 640)])

        def edge_pass(nedges, esrc_h, edst_h, tbl_h, do_deg):
            per_tile = nedges // NS
            nchunks = per_tile // K
            base0 = sid * per_tile

            def chunk(i, c):
                base = base0 + i * K
                pltpu.sync_copy(esrc_h.at[pl.ds(base, K)], src_v)
                pltpu.sync_copy(edst_h.at[pl.ds(base, K)], dst_v)
                if tbl_h is not None:
                    pltpu.async_copy(tbl_h.at[src_v], idx_v, sem).wait()
                    gidx = idx_v
                else:
                    gidx = src_v
                pltpu.async_copy(xw_h.at[gidx], rows_v, sem).wait()
                pltpu.sync_copy(rows_v, acc_sh.at[dst_v], add=True)
                if do_deg:
                    pltpu.sync_copy(ones_v, deg_sh.at[dst_v], add=True)
                return c
            lax.fori_loop(0, nchunks, chunk, 0)

        def copy_out(nrows_per_tile, agg_out, deg_out):
            r0 = sid * nrows_per_tile
            pltpu.sync_copy(acc_sh.at[pl.ds(r0, nrows_per_tile)],
                            agg_out.at[pl.ds(r0, nrows_per_tile)])
            if deg_out is not None:
                pltpu.sync_copy(deg_sh.at[pl.ds(sid * 640, 640)],
                                deg_out.at[pl.ds(sid * 640, 640)])

        zero_acc(N_PAD // NS)
        plsc.subcore_barrier()

        @pl.when(cid == 0)
        def _():
            edge_pass(E, src_h, dst_h, None, do_deg=True)

        @pl.when(cid == 1)
        def _():
            edge_pass(E, src_h, dst_h, perm_h, do_deg=False)

        plsc.subcore_barrier()

        @pl.when(cid == 0)
        def _():
            copy_out(N_PAD // NS, aggp_h, deg_h)

        @pl.when(cid == 1)
        def _():
            copy_out(N_PAD // NS, aggn_h, None)

        plsc.subcore_barrier()

        zero_acc(NA_PAD // NS)
        plsc.subcore_barrier()

        @pl.when(cid == 0)
        def _():
            edge_pass(EA, a1src_h, a1dst_h, a1n_h, do_deg=True)

        @pl.when(cid == 1)
        def _():
            edge_pass(EA, a2src_h, a2dst_h, a2n_h, do_deg=True)

        plsc.subcore_barrier()

        @pl.when(cid == 0)
        def _():
            copy_out(NA_PAD // NS, agg1_h, deg1_h)

        @pl.when(cid == 1)
        def _():
            copy_out(NA_PAD // NS, agg2_h, deg2_h)

    return k(xw, src, dst, perm, a1n, a1src, a1dst, a2n, a2src, a2dst)


def _mm_kernel(x_ref, w_ref, o_ref):
    o_ref[...] = jnp.dot(x_ref[...], w_ref[...],
                         preferred_element_type=jnp.float32)


def _summary_kernel(a1_ref, a2_ref, d1_ref, d2_ref, o_ref):
    @pl.when(pl.program_id(0) == 0)
    def _():
        o_ref[...] = jnp.zeros_like(o_ref)
    h1 = jnp.maximum(a1_ref[...], 0.0) / jnp.maximum(d1_ref[...], 1.0)
    h2 = jnp.maximum(a2_ref[...], 0.0) / jnp.maximum(d2_ref[...], 1.0)
    o_ref[0:1, :] += jnp.sum(h1, axis=0, keepdims=True)
    o_ref[1:2, :] += jnp.sum(h2, axis=0, keepdims=True)


def _loss_kernel(ap_ref, an_ref, deg_ref, ssum_ref, wd_ref, o_ref, *, nsteps):
    s = jax.nn.sigmoid(ssum_ref[...] / NA)
    dinv = 1.0 / jnp.maximum(deg_ref[...], 1.0)
    hp = jnp.maximum(ap_ref[...], 0.0) * dinv
    hn = jnp.maximum(an_ref[...], 0.0) * dinv
    dn = (((1,), (1,)), ((), ()))
    lp = lax.dot_general(jnp.dot(hp, wd_ref[...],
                                 preferred_element_type=jnp.float32), s, dn,
                         preferred_element_type=jnp.float32)
    ln = lax.dot_general(jnp.dot(hn, wd_ref[...],
                                 preferred_element_type=jnp.float32), s, dn,
                         preferred_element_type=jnp.float32)
    part = jnp.sum(jax.nn.softplus(lp) - lp) + jnp.sum(jax.nn.softplus(ln))

    @pl.when(pl.program_id(0) == 0)
    def _():
        o_ref[...] = jnp.zeros_like(o_ref)
    o_ref[...] += part.reshape(1, 1)

    @pl.when(pl.program_id(0) == nsteps - 1)
    def _():
        o_ref[...] = o_ref[...] / N


def kernel(x, edge_index, perm, aug1_nodes, aug1_edge_index, aug2_nodes,
           aug2_edge_index, W, Wd):
    xw = pl.pallas_call(
        _mm_kernel,
        grid=(10,),
        in_specs=[pl.BlockSpec((N // 10, D), lambda i: (i, 0)),
                  pl.BlockSpec((D, H), lambda i: (0, 0))],
        out_specs=pl.BlockSpec((N // 10, H), lambda i: (i, 0)),
        out_shape=jax.ShapeDtypeStruct((N, H), jnp.float32),
    )(x, W)

    src = edge_index[0]
    dst = edge_index[1]
    a1src = aug1_edge_index[0]
    a1dst = aug1_edge_index[1]
    a2src = aug2_edge_index[0]
    a2dst = aug2_edge_index[1]

    aggp, aggn, agg1, agg2, degp, deg1p, deg2p = _sc_segment_sums(
        xw, src, dst, perm, aug1_nodes, a1src, a1dst, aug2_nodes, a2src,
        a2dst)

    aggp = aggp[:N]
    aggn = aggn[:N]
    agg1 = agg1[:NA]
    agg2 = agg2[:NA]
    deg = degp[:N, None]
    deg1 = deg1p[:NA, None]
    deg2 = deg2p[:NA, None]

    ssum = pl.pallas_call(
        _summary_kernel,
        grid=(8,),
        in_specs=[pl.BlockSpec((NA // 8, H), lambda i: (i, 0)),
                  pl.BlockSpec((NA // 8, H), lambda i: (i, 0)),
                  pl.BlockSpec((NA // 8, 1), lambda i: (i, 0)),
                  pl.BlockSpec((NA // 8, 1), lambda i: (i, 0))],
        out_specs=pl.BlockSpec((2, H), lambda i: (0, 0)),
        out_shape=jax.ShapeDtypeStruct((2, H), jnp.float32),
    )(agg1, agg2, deg1, deg2)

    loss = pl.pallas_call(
        functools.partial(_loss_kernel, nsteps=10),
        grid=(10,),
        in_specs=[pl.BlockSpec((N // 10, H), lambda i: (i, 0)),
                  pl.BlockSpec((N // 10, H), lambda i: (i, 0)),
                  pl.BlockSpec((N // 10, 1), lambda i: (i, 0)),
                  pl.BlockSpec((2, H), lambda i: (0, 0)),
                  pl.BlockSpec((H, H), lambda i: (0, 0))],
        out_specs=pl.BlockSpec((1, 1), lambda i: (0, 0)),
        out_shape=jax.ShapeDtypeStruct((1, 1), jnp.float32),
    )(aggp, aggn, deg, ssum, Wd)

    return loss[0, 0]
